# baseline (speedup 1.0000x reference)
import functools
import math

import jax
import jax.numpy as jnp
from jax import lax
from jax.experimental import pallas as pl
from jax.experimental.pallas import tpu as pltpu
from jax.experimental.pallas import tpu_sc as plsc

D_MODEL = 64
SCALE = math.sqrt(D_MODEL)
LANES = 16
CH = 160


@functools.lru_cache(maxsize=None)
def _build(B):
    info = plsc.get_sparse_core_info()
    NC, NS = info.num_cores, info.num_subcores
    NW = NC * NS
    assert B % NW == 0
    b_per_w = B // NW
    assert b_per_w % CH == 0
    n_ch = b_per_w // CH
    assert n_ch % 2 == 0 and n_ch >= 6

    mesh = plsc.VectorSubcoreMesh(core_axis_name="c", subcore_axis_name="s")

    @functools.partial(
        pl.kernel,
        mesh=mesh,
        out_type=jax.ShapeDtypeStruct((B, D_MODEL), jnp.float32),
        compiler_params=pltpu.CompilerParams(use_tc_tiling_on_sc=True),
        scratch_types=[
            pltpu.VMEM((b_per_w,), jnp.int32),
            pltpu.VMEM((CH,), jnp.int32),
            pltpu.VMEM((CH,), jnp.int32),
            pltpu.VMEM((CH, 2 * D_MODEL), jnp.float32),
            pltpu.VMEM((CH, 2 * D_MODEL), jnp.float32),
            pltpu.VMEM((CH, D_MODEL), jnp.float32),
            pltpu.VMEM((CH, D_MODEL), jnp.float32),
            pltpu.SemaphoreType.DMA,
            pltpu.SemaphoreType.DMA,
            pltpu.SemaphoreType.DMA,
            pltpu.SemaphoreType.DMA,
        ],
    )
    def emb(idx_hbm, tab2_hbm, out_hbm, idx_v, pidx0, pidx1, gbuf0, gbuf1,
            obuf0, obuf1, gsem0, gsem1, osem0, osem1):
        pidx = (pidx0, pidx1)
        gbuf = (gbuf0, gbuf1)
        obuf = (obuf0, obuf1)
        gsem = (gsem0, gsem1)
        osem = (osem0, osem1)

        wid = lax.axis_index("s") * NC + lax.axis_index("c")
        base = wid * b_per_w
        pltpu.sync_copy(idx_hbm.at[pl.ds(base, b_per_w)], idx_v)

        def issue_gather(c, b):
            off = c * CH
            def mk(k, carry):
                sl = pl.ds(off + k * LANES, LANES)
                pidx[b][pl.ds(k * LANES, LANES)] = (
                    lax.shift_right_logical(idx_v[sl], 1)
                )
                return carry
            lax.fori_loop(0, CH // LANES, mk, 0)
            pltpu.async_copy(tab2_hbm.at[pidx[b]], gbuf[b], gsem[b])

        def wait_gather(b):
            pltpu.make_async_copy(tab2_hbm.at[pidx[b]], gbuf[b], gsem[b]).wait()

        def issue_out(c, b):
            pltpu.async_copy(
                obuf[b], out_hbm.at[pl.ds(base + c * CH, CH)], osem[b]
            )

        def wait_out(c, b):
            pltpu.make_async_copy(
                obuf[b], out_hbm.at[pl.ds(base + c * CH, CH)], osem[b]
            ).wait()

        def scale(c, b):
            off = c * CH
            def rows(g, carry):
                iv = idx_v[pl.ds(off + g * LANES, LANES)]
                pv = lax.mul(lax.bitwise_and(iv, 1), D_MODEL)
                for u in range(LANES):
                    r = g * LANES + u
                    p = pv[u]
                    for j in range(D_MODEL // LANES):
                        src = pl.ds(pl.multiple_of(p + j * LANES, LANES), LANES)
                        obuf[b][r, pl.ds(j * LANES, LANES)] = (
                            gbuf[b][r, src] * SCALE
                        )
                return carry
            lax.fori_loop(0, CH // LANES, rows, 0)

        issue_gather(0, 0)
        issue_gather(1, 1)
        for b in range(2):
            wait_gather(b)
            scale(b, b)
            issue_out(b, b)
            issue_gather(b + 2, b)

        def pair(i, carry):
            for b in range(2):
                c = 2 * i + b
                wait_gather(b)
                wait_out(c - 2, b)
                scale(c, b)
                issue_out(c, b)
                issue_gather(c + 2, b)
            return carry

        lax.fori_loop(1, n_ch // 2 - 1, pair, 0)

        for b in range(2):
            c = n_ch - 2 + b
            wait_gather(b)
            wait_out(c - 2, b)
            scale(c, b)
            issue_out(c, b)
        for b in range(2):
            wait_out(n_ch - 2 + b, b)

    return emb


def kernel(x, table):
    B = x.shape[0] * x.shape[1]
    idx = x.reshape(-1).astype(jnp.int32)
    tab2 = table.reshape(table.shape[0] // 2, 2 * D_MODEL)
    out = _build(B)(idx, tab2)
    return out.reshape(x.shape + (D_MODEL,))
